# baseline (speedup 1.0000x reference)
import functools

import jax
import jax.numpy as jnp
from jax import lax
from jax.experimental import pallas as pl
from jax.experimental.pallas import tpu as pltpu
from jax.experimental.pallas import tpu_sc as plsc

VOCAB = 1000000
MAXLEN = 200
EMBED = 64
BATCH = 4096

NC = 2
NS = 16
NW = NC * NS
BW = BATCH // NW
LPC = 2
NCHUNK = MAXLEN // LPC
LGRP = 8
GB = BW // 16


_mesh = plsc.VectorSubcoreMesh(core_axis_name="c", subcore_axis_name="s")


@functools.partial(
    pl.kernel,
    mesh=_mesh,
    out_type=jax.ShapeDtypeStruct((MAXLEN, EMBED, BATCH), jnp.float32),
    scratch_types=[
        pltpu.VMEM((BW, LGRP), jnp.int32),
        pltpu.VMEM((LGRP * BW,), jnp.int32),
        pltpu.VMEM((LGRP * BW,), jnp.int32),
        pltpu.VMEM((LPC * BW, 128), jnp.float32),
        pltpu.VMEM((LPC * BW, 128), jnp.float32),
        pltpu.VMEM((LPC, EMBED, BW), jnp.float32),
        pltpu.VMEM((LPC, EMBED, BW), jnp.float32),
        pltpu.VMEM((MAXLEN, EMBED), jnp.float32),
        pltpu.VMEM((EMBED, 16), jnp.float32),
        pltpu.SemaphoreType.DMA,
        pltpu.SemaphoreType.DMA,
        pltpu.SemaphoreType.DMA,
        pltpu.SemaphoreType.DMA,
    ],
    compiler_params=pltpu.CompilerParams(use_tc_tiling_on_sc=False,
                                         needs_layout_passes=False),
)
def _embed_kernel(x_hbm, tok_hbm, pos_hbm, out_hbm,
                  idxq, gidx, goff, gbuf0, gbuf1, obuf0, obuf1, pos_v, posx,
                  gsem0, gsem1, osem0, osem1):
    wid = lax.axis_index("s") * NC + lax.axis_index("c")
    b0 = wid * BW
    gbufs = (gbuf0, gbuf1)
    obufs = (obuf0, obuf1)
    gsems = (gsem0, gsem1)
    osems = (osem0, osem1)

    pltpu.sync_copy(pos_hbm, pos_v)

    lane = lax.iota(jnp.int32, 16)

    def stage_group(g):
        pltpu.sync_copy(x_hbm.at[pl.ds(b0, BW), pl.ds(g * LGRP, LGRP)], idxq)

        def stage_l(lp, carry):
            for bg in range(GB):
                row16 = lane + bg * 16
                col16 = jnp.full((16,), lp, jnp.int32)
                v = plsc.load_gather(idxq, [row16, col16])
                dst = pl.ds(lp * BW + bg * 16, 16)
                gidx[dst] = lax.shift_right_logical(v, 1)
                goff[dst] = lax.shift_left(lax.bitwise_and(v, 1), 6)
            return carry

        lax.fori_loop(0, LGRP, stage_l, 0)

    def issue_gather(c, slot):
        for h in range(LPC):
            lrow = (c % 4) * LPC + h
            pltpu.async_copy(
                tok_hbm.at[gidx.at[pl.ds(lrow * BW, BW)]],
                gbufs[slot].at[pl.ds(h * BW, BW)],
                gsems[slot],
            )

    def wait_gather(c, slot):
        for h in range(LPC):
            lrow = (c % 4) * LPC + h
            pltpu.make_async_copy(
                tok_hbm.at[gidx.at[pl.ds(lrow * BW, BW)]],
                gbufs[slot].at[pl.ds(h * BW, BW)],
                gsems[slot],
            ).wait()

    def issue_out(c, slot):
        pltpu.async_copy(
            obufs[slot],
            out_hbm.at[pl.ds(c * LPC, LPC), pl.ds(0, EMBED), pl.ds(b0, BW)],
            osems[slot],
        )

    def wait_out(slot):
        pltpu.make_async_copy(
            obufs[slot],
            out_hbm.at[pl.ds(0, LPC), pl.ds(0, EMBED), pl.ds(b0, BW)],
            osems[slot],
        ).wait()

    stage_group(0)
    issue_gather(0, 0)

    def quad_body(i, carry):
        for k in range(4):
            c = 4 * i + k
            slot = k % 2
            nslot = 1 - slot
            nxt = c + 1

            all_offs = [
                [goff[pl.ds((c % 4) * LPC * BW + h * BW + bg * 16, 16)]
                 for bg in range(GB)]
                for h in range(LPC)
            ]

            if k == 3:
                wait_gather(c, slot)

                @pl.when(nxt < NCHUNK)
                def _stage_next():
                    stage_group(i + 1)
                    issue_gather(nxt, nslot)
            else:
                @pl.when(nxt < NCHUNK)
                def _prefetch():
                    issue_gather(nxt, nslot)
                wait_gather(c, slot)

            @pl.when(c >= 2)
            def _reclaim():
                wait_out(slot)

            gbuf = gbufs[slot]
            obuf = obufs[slot]

            for h in range(LPC):
                l = c * LPC + h
                rows = [lane + (h * BW + bg * 16) for bg in range(GB)]
                offs = all_offs[h]

                def fill_pos(eq, carry2, l=l):
                    pq = pos_v[l, pl.ds(eq * 16, 16)]
                    for el in range(16):
                        posx[eq * 16 + el, pl.ds(0, 16)] = jnp.full(
                            (16,), pq[el], jnp.float32)
                    return carry2

                lax.fori_loop(0, EMBED // 16, fill_pos, 0)

                def emb_body(e, carry2, h=h, rows=rows, offs=offs,
                             gbuf=gbuf, obuf=obuf):
                    pv = posx[e, pl.ds(0, 16)]
                    for bg in range(GB):
                        col16 = offs[bg] + e
                        vals = plsc.load_gather(gbuf, [rows[bg], col16])
                        obuf[h, e, pl.ds(bg * 16, 16)] = vals + pv
                    return carry2

                lax.fori_loop(0, EMBED, emb_body, 0)

            issue_out(c, slot)
        return carry

    lax.fori_loop(0, NCHUNK // 4, quad_body, 0)
    wait_out(0)
    wait_out(1)


def kernel(x, token_table, pos_table):
    tok2 = token_table.reshape(VOCAB // 2, 128)
    out = _embed_kernel(x.astype(jnp.int32), tok2, pos_table)
    return jnp.transpose(out, (2, 0, 1))


# speedup vs baseline: 1.7078x; 1.7078x over previous
import functools

import jax
import jax.numpy as jnp
from jax import lax
from jax.experimental import pallas as pl
from jax.experimental.pallas import tpu as pltpu
from jax.experimental.pallas import tpu_sc as plsc

VOCAB = 1000000
MAXLEN = 200
EMBED = 64
BATCH = 4096

NC = 2
NS = 16
NW = NC * NS
BW = BATCH // NW
BPC = 2
NCHUNK = BW // BPC
SUBS = ((0, 96), (96, 104))


_mesh = plsc.VectorSubcoreMesh(core_axis_name="c", subcore_axis_name="s")


@functools.partial(
    pl.kernel,
    mesh=_mesh,
    out_type=jax.ShapeDtypeStruct((BATCH, MAXLEN, EMBED), jnp.float32),
    scratch_types=[
        pltpu.VMEM((2 * BPC, MAXLEN), jnp.int32),
        pltpu.VMEM((BPC, MAXLEN, EMBED), jnp.float32),
        pltpu.VMEM((BPC, MAXLEN, EMBED), jnp.float32),
        pltpu.VMEM((MAXLEN, EMBED), jnp.float32),
        pltpu.SemaphoreType.DMA,
        pltpu.SemaphoreType.DMA,
        pltpu.SemaphoreType.DMA,
        pltpu.SemaphoreType.DMA,
    ],
    compiler_params=pltpu.CompilerParams(use_tc_tiling_on_sc=False),
)
def _embed_kernel(x_hbm, tok_hbm, pos_hbm, out_hbm,
                  idx_v, buf0, buf1, pos_v, gsem0, gsem1, osem0, osem1):
    wid = lax.axis_index("s") * NC + lax.axis_index("c")
    batch_base = wid * BW
    bufs = (buf0, buf1)
    gsems = (gsem0, gsem1)
    osems = (osem0, osem1)

    pltpu.sync_copy(pos_hbm, pos_v)

    def gather_parts(c, slot):
        brow = batch_base + c * BPC
        parts = []
        for b in range(BPC):
            for off, size in SUBS:
                parts.append((
                    tok_hbm.at[idx_v.at[slot * BPC + b, pl.ds(off, size)]],
                    bufs[slot].at[b, pl.ds(off, size)],
                    gsems[slot],
                ))
        return brow, parts

    def issue_gather(c, slot):
        brow, parts = gather_parts(c, slot)
        pltpu.sync_copy(x_hbm.at[pl.ds(brow, BPC)],
                        idx_v.at[pl.ds(slot * BPC, BPC)])
        for src, dst, sem in parts:
            pltpu.async_copy(src, dst, sem)

    def wait_gather(c, slot):
        _, parts = gather_parts(c, slot)
        for src, dst, sem in parts:
            pltpu.make_async_copy(src, dst, sem).wait()

    def issue_out(c, slot):
        brow = batch_base + c * BPC
        pltpu.async_copy(bufs[slot], out_hbm.at[pl.ds(brow, BPC)], osems[slot])

    def wait_out(slot):
        pltpu.make_async_copy(bufs[slot], out_hbm.at[pl.ds(batch_base, BPC)],
                              osems[slot]).wait()

    issue_gather(0, 0)

    def pair_body(i, carry):
        c0 = 2 * i
        for slot in range(2):
            c = c0 + slot
            nslot = 1 - slot
            nxt = c + 1

            @pl.when(nxt < NCHUNK)
            def _prefetch():
                @pl.when(c >= 1)
                def _reclaim():
                    wait_out(nslot)
                issue_gather(nxt, nslot)

            wait_gather(c, slot)
            buf = bufs[slot]

            def pos_add(p, carry2):
                for q in range(EMBED // 16):
                    sl = pl.ds(q * 16, 16)
                    pv = pos_v[p, sl]
                    for b in range(BPC):
                        buf[b, p, sl] = buf[b, p, sl] + pv
                return carry2

            lax.fori_loop(0, MAXLEN, pos_add, 0)
            issue_out(c, slot)
        return carry

    lax.fori_loop(0, NCHUNK // 2, pair_body, 0)
    wait_out(0)
    wait_out(1)


def kernel(x, token_table, pos_table):
    return _embed_kernel(x.astype(jnp.int32), token_table, pos_table)
